# baseline (speedup 1.0000x reference)
import functools

import jax
import jax.numpy as jnp
from jax import lax
from jax.experimental import pallas as pl
from jax.experimental.pallas import tpu as pltpu
from jax.experimental.pallas import tpu_sc as plsc

NC = 2
NS = 16
NW = NC * NS

CH = 80
MBLK = 2000


def _transform(x, weight):
    n, d_in = x.shape
    r, _, d_out = weight.shape
    bm = 400
    nb = n // bm

    def body(x_ref, w_ref, o_ref):
        o_ref[...] = jnp.dot(x_ref[...], w_ref[0],
                             preferred_element_type=jnp.float32)

    return pl.pallas_call(
        body,
        grid=(r, nb),
        in_specs=[
            pl.BlockSpec((bm, d_in), lambda rr, i: (i, 0)),
            pl.BlockSpec((1, d_in, d_out), lambda rr, i: (rr, 0, 0)),
        ],
        out_specs=pl.BlockSpec((bm, d_out), lambda rr, i: (rr * nb + i, 0)),
        out_shape=jax.ShapeDtypeStruct((r * n, d_out), jnp.float32),
    )(x, weight)


def _finish(partials, bias):
    _, n, d_out = partials.shape
    bm = 400
    nb = n // bm

    def body(p_ref, b_ref, o_ref):
        o_ref[...] = p_ref[0] + p_ref[1] + b_ref[...]

    return pl.pallas_call(
        body,
        grid=(nb,),
        in_specs=[
            pl.BlockSpec((2, bm, d_out), lambda i: (0, i, 0)),
            pl.BlockSpec((1, d_out), lambda i: (0, 0)),
        ],
        out_specs=pl.BlockSpec((bm, d_out), lambda i: (i, 0)),
        out_shape=jax.ShapeDtypeStruct((n, d_out), jnp.float32),
    )(partials, bias.reshape(1, d_out))


def _sc_degree(dst, typ, n):
    e = dst.shape[0]
    kpad = 40960
    epw = e // NW
    krows = kpad // NS
    dzr = 1280

    mesh = plsc.VectorSubcoreMesh(
        core_axis_name="c", subcore_axis_name="s",
        num_cores=NC, num_subcores=NS)

    @functools.partial(
        pl.kernel,
        out_type=jax.ShapeDtypeStruct((NC * kpad, 16), jnp.float32),
        mesh=mesh,
        compiler_params=pltpu.CompilerParams(use_tc_tiling_on_sc=False),
        scratch_types=[
            pltpu.VMEM_SHARED((kpad, 16), jnp.float32),
            pltpu.VMEM((MBLK,), jnp.int32),
            pltpu.VMEM((MBLK,), jnp.int32),
            pltpu.VMEM((8, CH), jnp.int32),
            pltpu.VMEM((CH, 16), jnp.float32),
            pltpu.VMEM((dzr, 16), jnp.float32),
            pltpu.SemaphoreType.DMA,
        ],
    )
    def count(dst_hbm, typ_hbm, cnt_hbm,
              deg_sh, m_dst, m_typ, kidx, ones_v, zd, dsem):
        c = lax.axis_index("c")
        s = lax.axis_index("s")
        wid = s * NC + c
        zf = jnp.zeros((16,), jnp.float32)
        onef = jnp.full((16,), 1.0, jnp.float32)

        def z_zd(i, carry):
            zd[i, :] = zf
            return carry
        lax.fori_loop(0, dzr, z_zd, 0)

        def z_ones(i, carry):
            ones_v[i, :] = onef
            return carry
        lax.fori_loop(0, CH, z_ones, 0)

        for k in range(krows // dzr):
            pltpu.sync_copy(zd, deg_sh.at[pl.ds(s * krows + k * dzr, dzr)])
        plsc.subcore_barrier()

        nchb = MBLK // CH

        def drain_one():
            pltpu.make_async_copy(ones_v, deg_sh.at[kidx.at[0]],
                                  dsem).wait()

        def deg_block(b, carry):
            base = wid * epw + b * MBLK
            pltpu.sync_copy(dst_hbm.at[pl.ds(base, MBLK)], m_dst)
            pltpu.sync_copy(typ_hbm.at[pl.ds(base, MBLK)], m_typ)

            def deg_chunk(k, carry2):
                g_ch = b * nchb + k
                j = lax.rem(g_ch, 8)

                @pl.when(g_ch >= 8)
                def _drain():
                    drain_one()
                off = k * CH
                for g in range(CH // 16):
                    o = off + g * 16
                    d16 = m_dst[pl.ds(o, 16)]
                    t16 = m_typ[pl.ds(o, 16)]
                    kidx[j, pl.ds(g * 16, 16)] = t16 * n + d16
                pltpu.async_copy(ones_v, deg_sh.at[kidx.at[j]], dsem,
                                 add=True)
                return carry2
            lax.fori_loop(0, nchb, deg_chunk, 0)
            return carry
        lax.fori_loop(0, epw // MBLK, deg_block, 0)
        for _ in range(8):
            drain_one()
        plsc.subcore_barrier()

        for k in range(krows // dzr):
            r0 = s * krows + k * dzr
            pltpu.sync_copy(deg_sh.at[pl.ds(r0, dzr)],
                            cnt_hbm.at[pl.ds(c * kpad + r0, dzr)])

    @functools.partial(
        pl.kernel,
        out_type=jax.ShapeDtypeStruct((kpad, 16), jnp.float32),
        mesh=mesh,
        compiler_params=pltpu.CompilerParams(use_tc_tiling_on_sc=False),
        scratch_types=[
            pltpu.VMEM((dzr, 16), jnp.float32),
            pltpu.VMEM((dzr, 16), jnp.float32),
        ],
    )
    def merge(cnt_hbm, invrow_hbm, zd, zd2):
        c = lax.axis_index("c")
        s = lax.axis_index("s")
        r0 = (s * NC + c) * dzr
        pltpu.sync_copy(cnt_hbm.at[pl.ds(r0, dzr)], zd)
        pltpu.sync_copy(cnt_hbm.at[pl.ds(kpad + r0, dzr)], zd2)

        def inv_blk(i, carry):
            zd[i, :] = 1.0 / jnp.maximum(zd[i, :] + zd2[i, :], 1.0)
            return carry
        lax.fori_loop(0, dzr, inv_blk, 0)
        pltpu.sync_copy(zd, invrow_hbm.at[pl.ds(r0, dzr)])

    return merge(count(dst, typ))


def _sc_main(xw, src, dst, typ, invrow, n, d_out):
    e = src.shape[0]
    kpad = invrow.shape[0]
    epw = e // NW
    orows = 624
    otail = n - orows * NS
    zrows = 48

    mesh = plsc.VectorSubcoreMesh(
        core_axis_name="c", subcore_axis_name="s",
        num_cores=NC, num_subcores=NS)

    nch = epw // CH

    @functools.partial(
        pl.kernel,
        out_type=jax.ShapeDtypeStruct((NC, n, d_out), jnp.float32),
        mesh=mesh,
        compiler_params=pltpu.CompilerParams(use_tc_tiling_on_sc=False),
        scratch_types=[
            pltpu.VMEM_SHARED((n, d_out), jnp.float32),
            pltpu.VMEM((MBLK,), jnp.int32),
            pltpu.VMEM((MBLK,), jnp.int32),
            pltpu.VMEM((MBLK,), jnp.int32),
            [pltpu.VMEM((CH, 128), jnp.float32)] * 3,
            [pltpu.VMEM((CH, 16), jnp.float32)] * 3,
            [pltpu.VMEM((CH,), jnp.int32)] * 3,
            [pltpu.VMEM((CH,), jnp.int32)] * 3,
            [pltpu.VMEM((CH,), jnp.int32)] * 3,
            pltpu.VMEM((zrows, 128), jnp.float32),
            [pltpu.SemaphoreType.DMA] * 3,
            [pltpu.SemaphoreType.DMA] * 3,
            [pltpu.SemaphoreType.DMA] * 3,
        ],
    )
    def agg(xw_hbm, src_hbm, dst_hbm, typ_hbm, invrow_hbm, pout_hbm,
            out_sh, m_src, m_dst, m_typ,
            rows_v, rinv, gidx, didx, kidx, zb, sem0, sem1, sem2):
        c = lax.axis_index("c")
        s = lax.axis_index("s")
        wid = s * NC + c
        zf = jnp.zeros((16,), jnp.float32)

        def z_zb(i, carry):
            zb[i // 8, pl.ds((i % 8) * 16, 16)] = zf
            return carry
        lax.fori_loop(0, zrows * 8, z_zb, 0)

        for k in range(orows // zrows):
            pltpu.sync_copy(zb, out_sh.at[pl.ds(s * orows + k * zrows, zrows)])

        @pl.when(s == NS - 1)
        def _zero_tail():
            pltpu.sync_copy(zb.at[pl.ds(0, otail)],
                            out_sh.at[pl.ds(orows * NS, otail)])
        plsc.subcore_barrier()

        def make_idx(k, buf):
            off = k * CH
            for g in range(CH // 16):
                o = off + g * 16
                s16 = m_src[pl.ds(o, 16)]
                d16 = m_dst[pl.ds(o, 16)]
                t16 = m_typ[pl.ds(o, 16)]
                tn = t16 * n
                gidx[buf][pl.ds(g * 16, 16)] = tn + s16
                didx[buf][pl.ds(g * 16, 16)] = d16
                kidx[buf][pl.ds(g * 16, 16)] = tn + d16

        def start_gather(buf):
            cp0 = pltpu.async_copy(xw_hbm.at[gidx[buf]], rows_v[buf], sem0[buf])
            cp1 = pltpu.async_copy(invrow_hbm.at[kidx[buf]], rinv[buf],
                                   sem1[buf])
            return cp0, cp1

        def drain_gather(buf):
            pltpu.make_async_copy(xw_hbm.at[gidx[buf]], rows_v[buf],
                                  sem0[buf]).wait()
            pltpu.make_async_copy(invrow_hbm.at[kidx[buf]], rinv[buf],
                                  sem1[buf]).wait()

        def drain_scatter(buf):
            pltpu.make_async_copy(rows_v[buf], out_sh.at[didx[buf]],
                                  sem2[buf]).wait()

        def scale_scatter(buf):
            def scale(i, carry3):
                for u in range(4):
                    j = i * 4 + u
                    scv = rinv[buf][j, :]
                    for q in range(8):
                        rows_v[buf][j, pl.ds(q * 16, 16)] = (
                            rows_v[buf][j, pl.ds(q * 16, 16)] * scv)
                return carry3
            lax.fori_loop(0, CH // 4, scale, 0)
            pltpu.async_copy(rows_v[buf], out_sh.at[didx[buf]], sem2[buf],
                             add=True)

        nchb = MBLK // CH

        def main_block(b, carry):
            base = wid * epw + b * MBLK
            pltpu.sync_copy(src_hbm.at[pl.ds(base, MBLK)], m_src)
            pltpu.sync_copy(dst_hbm.at[pl.ds(base, MBLK)], m_dst)
            pltpu.sync_copy(typ_hbm.at[pl.ds(base, MBLK)], m_typ)

            for k0 in range(2):
                make_idx(k0, k0)
                start_gather(k0)

            def step(k, carry2):
                jk = lax.rem(k, 3)
                for buf in range(3):
                    @pl.when(jk == buf)
                    def _proc(buf=buf):
                        drain_gather(buf)
                        scale_scatter(buf)

                @pl.when(k <= nchb - 3)
                def _prefetch():
                    jp = lax.rem(k + 2, 3)
                    for buf in range(3):
                        @pl.when(jp == buf)
                        def _pref(buf=buf):
                            @pl.when(k >= 1)
                            def _drain_prev():
                                drain_scatter(buf)
                            make_idx(k + 2, buf)
                            start_gather(buf)
                return carry2
            lax.fori_loop(0, nchb, step, 0)
            for buf in range(3):
                drain_scatter(buf)
            return carry
        lax.fori_loop(0, epw // MBLK, main_block, 0)
        plsc.subcore_barrier()

        pltpu.sync_copy(out_sh.at[pl.ds(s * orows, orows)],
                        pout_hbm.at[c, pl.ds(s * orows, orows)])

        @pl.when(s == NS - 1)
        def _write_tail():
            pltpu.sync_copy(out_sh.at[pl.ds(orows * NS, otail)],
                            pout_hbm.at[c, pl.ds(orows * NS, otail)])

    return agg(xw, src, dst, typ, invrow)


def kernel(x, edge_index, edge_type, weight, bias):
    n, _ = x.shape
    d_out = weight.shape[2]
    src = edge_index[0]
    dst = edge_index[1]
    xw = _transform(x, weight)
    invrow = _sc_degree(dst, edge_type, n)
    partials = _sc_main(xw, src, dst, edge_type, invrow, n, d_out)
    return _finish(partials, bias)


# speedup vs baseline: 1.0901x; 1.0901x over previous
import functools

import jax
import jax.numpy as jnp
from jax import lax
from jax.experimental import pallas as pl
from jax.experimental.pallas import tpu as pltpu
from jax.experimental.pallas import tpu_sc as plsc

NC = 2
NS = 16
NW = NC * NS

CH = 80
MBLK = 2000


def _transform(x, weight):
    n, d_in = x.shape
    r, _, d_out = weight.shape
    bm = 400
    nb = n // bm

    def body(x_ref, w_ref, o_ref):
        o_ref[...] = jnp.dot(x_ref[...], w_ref[0],
                             preferred_element_type=jnp.float32)

    return pl.pallas_call(
        body,
        grid=(r, nb),
        in_specs=[
            pl.BlockSpec((bm, d_in), lambda rr, i: (i, 0)),
            pl.BlockSpec((1, d_in, d_out), lambda rr, i: (rr, 0, 0)),
        ],
        out_specs=pl.BlockSpec((bm, d_out), lambda rr, i: (rr * nb + i, 0)),
        out_shape=jax.ShapeDtypeStruct((r * n, d_out), jnp.float32),
    )(x, weight)


def _finish(partials, bias):
    _, n, d_out = partials.shape
    bm = 400
    nb = n // bm

    def body(p_ref, b_ref, o_ref):
        o_ref[...] = p_ref[0] + p_ref[1] + b_ref[...]

    return pl.pallas_call(
        body,
        grid=(nb,),
        in_specs=[
            pl.BlockSpec((2, bm, d_out), lambda i: (0, i, 0)),
            pl.BlockSpec((1, d_out), lambda i: (0, 0)),
        ],
        out_specs=pl.BlockSpec((bm, d_out), lambda i: (i, 0)),
        out_shape=jax.ShapeDtypeStruct((n, d_out), jnp.float32),
    )(partials, bias.reshape(1, d_out))


def _sc_degree(dst, typ, n):
    e = dst.shape[0]
    kpad = 40960
    eps = e // NS
    krows = kpad // NS
    dzr = 1280

    mesh = plsc.VectorSubcoreMesh(
        core_axis_name="c", subcore_axis_name="s",
        num_cores=NC, num_subcores=NS)

    @functools.partial(
        pl.kernel,
        out_type=jax.ShapeDtypeStruct((kpad, 16), jnp.float32),
        mesh=mesh,
        compiler_params=pltpu.CompilerParams(use_tc_tiling_on_sc=False),
        scratch_types=[
            pltpu.VMEM_SHARED((kpad, 16), jnp.float32),
            pltpu.VMEM((MBLK,), jnp.int32),
            pltpu.VMEM((MBLK,), jnp.int32),
            pltpu.VMEM((8, CH), jnp.int32),
            pltpu.VMEM((CH, 16), jnp.float32),
            pltpu.VMEM((dzr, 16), jnp.float32),
            pltpu.SemaphoreType.DMA,
        ],
    )
    def count(dst_hbm, typ_hbm, invrow_hbm,
              deg_sh, m_dst, m_typ, kidx, ones_v, zd, dsem):
        c = lax.axis_index("c")
        s = lax.axis_index("s")
        zf = jnp.zeros((16,), jnp.float32)
        onef = jnp.full((16,), 1.0, jnp.float32)

        def z_zd(i, carry):
            zd[i, :] = zf
            return carry
        lax.fori_loop(0, dzr, z_zd, 0)

        def z_ones(i, carry):
            ones_v[i, :] = onef
            return carry
        lax.fori_loop(0, CH, z_ones, 0)

        for k in range(krows // dzr):
            pltpu.sync_copy(zd, deg_sh.at[pl.ds(s * krows + k * dzr, dzr)])
        plsc.subcore_barrier()

        nchb = MBLK // CH

        def drain_one():
            pltpu.make_async_copy(ones_v, deg_sh.at[kidx.at[0]],
                                  dsem).wait()

        def deg_block(b, carry):
            base = s * eps + b * MBLK
            pltpu.sync_copy(dst_hbm.at[pl.ds(base, MBLK)], m_dst)
            pltpu.sync_copy(typ_hbm.at[pl.ds(base, MBLK)], m_typ)

            def deg_chunk(k, carry2):
                g_ch = b * nchb + k
                j = lax.rem(g_ch, 8)

                @pl.when(g_ch >= 8)
                def _drain():
                    drain_one()
                off = k * CH
                for g in range(CH // 16):
                    o = off + g * 16
                    d16 = m_dst[pl.ds(o, 16)]
                    t16 = m_typ[pl.ds(o, 16)]
                    kidx[j, pl.ds(g * 16, 16)] = t16 * n + d16
                pltpu.async_copy(ones_v, deg_sh.at[kidx.at[j]], dsem,
                                 add=True)
                return carry2
            lax.fori_loop(0, nchb, deg_chunk, 0)
            return carry
        lax.fori_loop(0, eps // MBLK, deg_block, 0)
        for _ in range(8):
            drain_one()
        plsc.subcore_barrier()

        r0 = c * (kpad // NC) + s * dzr
        pltpu.sync_copy(deg_sh.at[pl.ds(r0, dzr)], zd)

        def inv_blk(i, carry):
            zd[i, :] = 1.0 / jnp.maximum(zd[i, :], 1.0)
            return carry
        lax.fori_loop(0, dzr, inv_blk, 0)
        pltpu.sync_copy(zd, invrow_hbm.at[pl.ds(r0, dzr)])

    return count(dst, typ)


def _sc_main(xw, src, dst, typ, invrow, n, d_out):
    e = src.shape[0]
    kpad = invrow.shape[0]
    epw = e // NW
    orows = 624
    otail = n - orows * NS
    zrows = 48

    mesh = plsc.VectorSubcoreMesh(
        core_axis_name="c", subcore_axis_name="s",
        num_cores=NC, num_subcores=NS)

    nch = epw // CH

    @functools.partial(
        pl.kernel,
        out_type=jax.ShapeDtypeStruct((NC, n, d_out), jnp.float32),
        mesh=mesh,
        compiler_params=pltpu.CompilerParams(use_tc_tiling_on_sc=False),
        scratch_types=[
            pltpu.VMEM_SHARED((n, d_out), jnp.float32),
            pltpu.VMEM((MBLK,), jnp.int32),
            pltpu.VMEM((MBLK,), jnp.int32),
            pltpu.VMEM((MBLK,), jnp.int32),
            [pltpu.VMEM((CH, 128), jnp.float32)] * 3,
            [pltpu.VMEM((CH, 16), jnp.float32)] * 3,
            [pltpu.VMEM((CH,), jnp.int32)] * 3,
            [pltpu.VMEM((CH,), jnp.int32)] * 3,
            [pltpu.VMEM((CH,), jnp.int32)] * 3,
            pltpu.VMEM((zrows, 128), jnp.float32),
            [pltpu.SemaphoreType.DMA] * 3,
            [pltpu.SemaphoreType.DMA] * 3,
            [pltpu.SemaphoreType.DMA] * 3,
        ],
    )
    def agg(xw_hbm, src_hbm, dst_hbm, typ_hbm, invrow_hbm, pout_hbm,
            out_sh, m_src, m_dst, m_typ,
            rows_v, rinv, gidx, didx, kidx, zb, sem0, sem1, sem2):
        c = lax.axis_index("c")
        s = lax.axis_index("s")
        wid = s * NC + c
        zf = jnp.zeros((16,), jnp.float32)

        def z_zb(i, carry):
            zb[i // 8, pl.ds((i % 8) * 16, 16)] = zf
            return carry
        lax.fori_loop(0, zrows * 8, z_zb, 0)

        for k in range(orows // zrows):
            pltpu.sync_copy(zb, out_sh.at[pl.ds(s * orows + k * zrows, zrows)])

        @pl.when(s == NS - 1)
        def _zero_tail():
            pltpu.sync_copy(zb.at[pl.ds(0, otail)],
                            out_sh.at[pl.ds(orows * NS, otail)])
        plsc.subcore_barrier()

        def make_idx(k, buf):
            off = k * CH
            for g in range(CH // 16):
                o = off + g * 16
                s16 = m_src[pl.ds(o, 16)]
                d16 = m_dst[pl.ds(o, 16)]
                t16 = m_typ[pl.ds(o, 16)]
                tn = t16 * n
                gidx[buf][pl.ds(g * 16, 16)] = tn + s16
                didx[buf][pl.ds(g * 16, 16)] = d16
                kidx[buf][pl.ds(g * 16, 16)] = tn + d16

        def start_gather(buf):
            cp0 = pltpu.async_copy(xw_hbm.at[gidx[buf]], rows_v[buf], sem0[buf])
            cp1 = pltpu.async_copy(invrow_hbm.at[kidx[buf]], rinv[buf],
                                   sem1[buf])
            return cp0, cp1

        def drain_gather(buf):
            pltpu.make_async_copy(xw_hbm.at[gidx[buf]], rows_v[buf],
                                  sem0[buf]).wait()
            pltpu.make_async_copy(invrow_hbm.at[kidx[buf]], rinv[buf],
                                  sem1[buf]).wait()

        def drain_scatter(buf):
            pltpu.make_async_copy(rows_v[buf], out_sh.at[didx[buf]],
                                  sem2[buf]).wait()

        def scale_scatter(buf):
            def scale(i, carry3):
                for u in range(4):
                    j = i * 4 + u
                    scv = rinv[buf][j, :]
                    for q in range(8):
                        rows_v[buf][j, pl.ds(q * 16, 16)] = (
                            rows_v[buf][j, pl.ds(q * 16, 16)] * scv)
                return carry3
            lax.fori_loop(0, CH // 4, scale, 0)
            pltpu.async_copy(rows_v[buf], out_sh.at[didx[buf]], sem2[buf],
                             add=True)

        nchb = MBLK // CH

        def main_block(b, carry):
            base = wid * epw + b * MBLK
            pltpu.sync_copy(src_hbm.at[pl.ds(base, MBLK)], m_src)
            pltpu.sync_copy(dst_hbm.at[pl.ds(base, MBLK)], m_dst)
            pltpu.sync_copy(typ_hbm.at[pl.ds(base, MBLK)], m_typ)

            for k0 in range(2):
                make_idx(k0, k0)
                start_gather(k0)

            def step(k, carry2):
                jk = lax.rem(k, 3)
                for buf in range(3):
                    @pl.when(jk == buf)
                    def _proc(buf=buf):
                        drain_gather(buf)
                        scale_scatter(buf)

                @pl.when(k <= nchb - 3)
                def _prefetch():
                    jp = lax.rem(k + 2, 3)
                    for buf in range(3):
                        @pl.when(jp == buf)
                        def _pref(buf=buf):
                            @pl.when(k >= 1)
                            def _drain_prev():
                                drain_scatter(buf)
                            make_idx(k + 2, buf)
                            start_gather(buf)
                return carry2
            lax.fori_loop(0, nchb, step, 0)
            for buf in range(3):
                drain_scatter(buf)
            return carry
        lax.fori_loop(0, epw // MBLK, main_block, 0)
        plsc.subcore_barrier()

        pltpu.sync_copy(out_sh.at[pl.ds(s * orows, orows)],
                        pout_hbm.at[c, pl.ds(s * orows, orows)])

        @pl.when(s == NS - 1)
        def _write_tail():
            pltpu.sync_copy(out_sh.at[pl.ds(orows * NS, otail)],
                            pout_hbm.at[c, pl.ds(orows * NS, otail)])

    return agg(xw, src, dst, typ, invrow)


def kernel(x, edge_index, edge_type, weight, bias):
    n, _ = x.shape
    d_out = weight.shape[2]
    src = edge_index[0]
    dst = edge_index[1]
    xw = _transform(x, weight)
    invrow = _sc_degree(dst, edge_type, n)
    partials = _sc_main(xw, src, dst, edge_type, invrow, n, d_out)
    return _finish(partials, bias)


# speedup vs baseline: 1.1823x; 1.0846x over previous
import functools

import jax
import jax.numpy as jnp
from jax import lax
from jax.experimental import pallas as pl
from jax.experimental.pallas import tpu as pltpu
from jax.experimental.pallas import tpu_sc as plsc

NC = 2
NS = 16
NW = NC * NS

CH = 80
MBLK = 2000


def _transform(x, weight):
    n, d_in = x.shape
    r, _, d_out = weight.shape
    bm = 1000
    nb = n // bm

    def body(x_ref, w_ref, o_ref):
        o_ref[...] = jnp.dot(x_ref[...], w_ref[0],
                             preferred_element_type=jnp.float32)

    return pl.pallas_call(
        body,
        grid=(r, nb),
        in_specs=[
            pl.BlockSpec((bm, d_in), lambda rr, i: (i, 0)),
            pl.BlockSpec((1, d_in, d_out), lambda rr, i: (rr, 0, 0)),
        ],
        out_specs=pl.BlockSpec((bm, d_out), lambda rr, i: (rr * nb + i, 0)),
        out_shape=jax.ShapeDtypeStruct((r * n, d_out), jnp.float32),
    )(x, weight)


def _finish(partials, bias):
    _, n, d_out = partials.shape
    bm = 1000
    nb = n // bm

    def body(p_ref, b_ref, o_ref):
        o_ref[...] = p_ref[0] + p_ref[1] + b_ref[...]

    return pl.pallas_call(
        body,
        grid=(nb,),
        in_specs=[
            pl.BlockSpec((2, bm, d_out), lambda i: (0, i, 0)),
            pl.BlockSpec((1, d_out), lambda i: (0, 0)),
        ],
        out_specs=pl.BlockSpec((bm, d_out), lambda i: (i, 0)),
        out_shape=jax.ShapeDtypeStruct((n, d_out), jnp.float32),
    )(partials, bias.reshape(1, d_out))


def _sc_degree(dst, typ, n):
    e = dst.shape[0]
    kpad = 40960
    eps = e // NS
    krows = kpad // NS
    dzr = 1280

    mesh = plsc.VectorSubcoreMesh(
        core_axis_name="c", subcore_axis_name="s",
        num_cores=NC, num_subcores=NS)

    @functools.partial(
        pl.kernel,
        out_type=jax.ShapeDtypeStruct((kpad, 16), jnp.float32),
        mesh=mesh,
        compiler_params=pltpu.CompilerParams(use_tc_tiling_on_sc=False),
        scratch_types=[
            pltpu.VMEM_SHARED((kpad, 16), jnp.float32),
            pltpu.VMEM((MBLK,), jnp.int32),
            pltpu.VMEM((MBLK,), jnp.int32),
            pltpu.VMEM((8, CH), jnp.int32),
            pltpu.VMEM((CH, 16), jnp.float32),
            pltpu.VMEM((dzr, 16), jnp.float32),
            pltpu.SemaphoreType.DMA,
        ],
    )
    def count(dst_hbm, typ_hbm, invrow_hbm,
              deg_sh, m_dst, m_typ, kidx, ones_v, zd, dsem):
        c = lax.axis_index("c")
        s = lax.axis_index("s")
        zf = jnp.zeros((16,), jnp.float32)
        onef = jnp.full((16,), 1.0, jnp.float32)

        def z_zd(i, carry):
            zd[i, :] = zf
            return carry
        lax.fori_loop(0, dzr, z_zd, 0)

        def z_ones(i, carry):
            ones_v[i, :] = onef
            return carry
        lax.fori_loop(0, CH, z_ones, 0)

        for k in range(krows // dzr):
            pltpu.sync_copy(zd, deg_sh.at[pl.ds(s * krows + k * dzr, dzr)])
        plsc.subcore_barrier()

        nchb = MBLK // CH

        def drain_one():
            pltpu.make_async_copy(ones_v, deg_sh.at[kidx.at[0]],
                                  dsem).wait()

        def deg_block(b, carry):
            base = s * eps + b * MBLK
            pltpu.sync_copy(dst_hbm.at[pl.ds(base, MBLK)], m_dst)
            pltpu.sync_copy(typ_hbm.at[pl.ds(base, MBLK)], m_typ)

            def deg_chunk(k, carry2):
                g_ch = b * nchb + k
                j = lax.rem(g_ch, 8)

                @pl.when(g_ch >= 8)
                def _drain():
                    drain_one()
                off = k * CH
                for g in range(CH // 16):
                    o = off + g * 16
                    d16 = m_dst[pl.ds(o, 16)]
                    t16 = m_typ[pl.ds(o, 16)]
                    kidx[j, pl.ds(g * 16, 16)] = t16 * n + d16
                pltpu.async_copy(ones_v, deg_sh.at[kidx.at[j]], dsem,
                                 add=True)
                return carry2
            lax.fori_loop(0, nchb, deg_chunk, 0)
            return carry
        lax.fori_loop(0, eps // MBLK, deg_block, 0)
        for _ in range(8):
            drain_one()
        plsc.subcore_barrier()

        r0 = c * (kpad // NC) + s * dzr
        pltpu.sync_copy(deg_sh.at[pl.ds(r0, dzr)], zd)

        def inv_blk(i, carry):
            zd[i, :] = 1.0 / jnp.maximum(zd[i, :], 1.0)
            return carry
        lax.fori_loop(0, dzr, inv_blk, 0)
        pltpu.sync_copy(zd, invrow_hbm.at[pl.ds(r0, dzr)])

    return count(dst, typ)


def _sc_main(xw, src, dst, typ, invrow, n, d_out):
    e = src.shape[0]
    kpad = invrow.shape[0]
    epw = e // NW
    orows = 624
    otail = n - orows * NS
    zrows = 48

    mesh = plsc.VectorSubcoreMesh(
        core_axis_name="c", subcore_axis_name="s",
        num_cores=NC, num_subcores=NS)

    @functools.partial(
        pl.kernel,
        out_type=jax.ShapeDtypeStruct((NC, n, d_out), jnp.float32),
        mesh=mesh,
        compiler_params=pltpu.CompilerParams(use_tc_tiling_on_sc=False),
        scratch_types=[
            pltpu.VMEM_SHARED((n, d_out), jnp.float32),
            pltpu.VMEM((MBLK,), jnp.int32),
            pltpu.VMEM((MBLK,), jnp.int32),
            pltpu.VMEM((MBLK,), jnp.int32),
            [pltpu.VMEM((CH, 128), jnp.float32)] * 3,
            [pltpu.VMEM((CH, 16), jnp.float32)] * 3,
            [pltpu.VMEM((CH,), jnp.int32)] * 3,
            [pltpu.VMEM((CH,), jnp.int32)] * 3,
            [pltpu.VMEM((CH,), jnp.int32)] * 3,
            pltpu.VMEM((zrows, 128), jnp.float32),
            [pltpu.SemaphoreType.DMA] * 3,
            [pltpu.SemaphoreType.DMA] * 3,
            [pltpu.SemaphoreType.DMA] * 3,
        ],
    )
    def agg(xw_hbm, src_hbm, dst_hbm, typ_hbm, invrow_hbm, pout_hbm,
            out_sh, m_src, m_dst, m_typ,
            rows_v, rinv, gidx, didx, kidx, zb, sem0, sem1, sem2):
        c = lax.axis_index("c")
        s = lax.axis_index("s")
        wid = s * NC + c
        zf = jnp.zeros((16,), jnp.float32)

        def z_zb(i, carry):
            zb[i // 8, pl.ds((i % 8) * 16, 16)] = zf
            return carry
        lax.fori_loop(0, zrows * 8, z_zb, 0)

        for k in range(orows // zrows):
            pltpu.sync_copy(zb, out_sh.at[pl.ds(s * orows + k * zrows, zrows)])

        @pl.when(s == NS - 1)
        def _zero_tail():
            pltpu.sync_copy(zb.at[pl.ds(0, otail)],
                            out_sh.at[pl.ds(orows * NS, otail)])
        plsc.subcore_barrier()

        def make_idx(k, buf):
            off = k * CH
            for g in range(CH // 16):
                o = off + g * 16
                s16 = m_src[pl.ds(o, 16)]
                d16 = m_dst[pl.ds(o, 16)]
                t16 = m_typ[pl.ds(o, 16)]
                tn = t16 * n
                gidx[buf][pl.ds(g * 16, 16)] = tn + s16
                didx[buf][pl.ds(g * 16, 16)] = d16
                kidx[buf][pl.ds(g * 16, 16)] = tn + d16

        def start_gather(buf):
            cp0 = pltpu.async_copy(xw_hbm.at[gidx[buf]], rows_v[buf], sem0[buf])
            cp1 = pltpu.async_copy(invrow_hbm.at[kidx[buf]], rinv[buf],
                                   sem1[buf])
            return cp0, cp1

        def drain_gather(buf):
            pltpu.make_async_copy(xw_hbm.at[gidx[buf]], rows_v[buf],
                                  sem0[buf]).wait()
            pltpu.make_async_copy(invrow_hbm.at[kidx[buf]], rinv[buf],
                                  sem1[buf]).wait()

        def drain_scatter(buf):
            pltpu.make_async_copy(rows_v[buf], out_sh.at[didx[buf]],
                                  sem2[buf]).wait()

        def scale_scatter(buf):
            def scale(i, carry3):
                for u in range(4):
                    j = i * 4 + u
                    scv = rinv[buf][j, :]
                    for q in range(8):
                        rows_v[buf][j, pl.ds(q * 16, 16)] = (
                            rows_v[buf][j, pl.ds(q * 16, 16)] * scv)
                return carry3
            lax.fori_loop(0, CH // 4, scale, 0)
            pltpu.async_copy(rows_v[buf], out_sh.at[didx[buf]], sem2[buf],
                             add=True)

        nchb = MBLK // CH

        def main_block(b, carry):
            base = wid * epw + b * MBLK
            pltpu.sync_copy(src_hbm.at[pl.ds(base, MBLK)], m_src)
            pltpu.sync_copy(dst_hbm.at[pl.ds(base, MBLK)], m_dst)
            pltpu.sync_copy(typ_hbm.at[pl.ds(base, MBLK)], m_typ)

            for k0 in range(2):
                make_idx(k0, k0)
                start_gather(k0)

            def step(k, carry2):
                jk = lax.rem(k, 3)
                for buf in range(3):
                    @pl.when(jk == buf)
                    def _proc(buf=buf):
                        drain_gather(buf)
                        scale_scatter(buf)

                @pl.when(k <= nchb - 3)
                def _prefetch():
                    jp = lax.rem(k + 2, 3)
                    for buf in range(3):
                        @pl.when(jp == buf)
                        def _pref(buf=buf):
                            @pl.when(k >= 1)
                            def _drain_prev():
                                drain_scatter(buf)
                            make_idx(k + 2, buf)
                            start_gather(buf)
                return carry2
            lax.fori_loop(0, nchb, step, 0)
            for buf in range(3):
                drain_scatter(buf)
            return carry
        lax.fori_loop(0, epw // MBLK, main_block, 0)
        plsc.subcore_barrier()

        pltpu.sync_copy(out_sh.at[pl.ds(s * orows, orows)],
                        pout_hbm.at[c, pl.ds(s * orows, orows)])

        @pl.when(s == NS - 1)
        def _write_tail():
            pltpu.sync_copy(out_sh.at[pl.ds(orows * NS, otail)],
                            pout_hbm.at[c, pl.ds(orows * NS, otail)])

    return agg(xw, src, dst, typ, invrow)


def kernel(x, edge_index, edge_type, weight, bias):
    n, _ = x.shape
    d_out = weight.shape[2]
    src = edge_index[0]
    dst = edge_index[1]
    xw = _transform(x, weight)
    invrow = _sc_degree(dst, edge_type, n)
    partials = _sc_main(xw, src, dst, edge_type, invrow, n, d_out)
    return _finish(partials, bias)
